# baseline (speedup 1.0000x reference)
import functools

import jax
import jax.numpy as jnp
from jax.experimental import pallas as pl
from jax.experimental.pallas import tpu as pltpu


def _matmul_kernel(a_ref, b_ref, o_ref, acc_ref, *, k_total, bk, nk):
    k = pl.program_id(1)

    @pl.when(k == 0)
    def _init():
        acc_ref[...] = jnp.zeros_like(acc_ref)

    a = a_ref[...]
    b = b_ref[...]
    kbase = k * bk
    rem = k_total - kbase
    col_ids = jax.lax.broadcasted_iota(jnp.int32, (1, bk), 1)
    row_ids = jax.lax.broadcasted_iota(jnp.int32, (bk, 1), 0)
    a = jnp.where(col_ids < rem, a, 0.0)
    b = jnp.where(row_ids < rem, b, 0.0)
    acc_ref[...] += jnp.dot(
        a.astype(jnp.bfloat16),
        b.astype(jnp.bfloat16),
        preferred_element_type=jnp.float32,
    )

    @pl.when(k == nk - 1)
    def _store():
        o_ref[...] = acc_ref[...]


def _matmul_linear_kernel(a_ref, b_ref, w_ref, bias_ref, o_ref, acc_ref,
                          *, k_total, bk, nk):
    k = pl.program_id(1)

    @pl.when(k == 0)
    def _init():
        acc_ref[...] = jnp.zeros_like(acc_ref)

    a = a_ref[...]
    b = b_ref[...]
    kbase = k * bk
    rem = k_total - kbase
    col_ids = jax.lax.broadcasted_iota(jnp.int32, (1, bk), 1)
    row_ids = jax.lax.broadcasted_iota(jnp.int32, (bk, 1), 0)
    a = jnp.where(col_ids < rem, a, 0.0)
    b = jnp.where(row_ids < rem, b, 0.0)
    acc_ref[...] += jnp.dot(
        a.astype(jnp.bfloat16),
        b.astype(jnp.bfloat16),
        preferred_element_type=jnp.float32,
    )

    @pl.when(k == nk - 1)
    def _store():
        acc = acc_ref[...].astype(jnp.bfloat16)
        w = w_ref[...].astype(jnp.bfloat16)
        out = jax.lax.dot_general(
            acc, w, (((1,), (1,)), ((), ())),
            preferred_element_type=jnp.float32,
        )
        o_ref[...] = out + bias_ref[...]


def _pick_bm(m):
    for cand in (2000, 1024, 512, 256, 128, 64, 32, 16, 8):
        if m % cand == 0:
            return cand
    return m


def _propagate(adj, rhs, w=None, bias=None, bm=None, bk=2048):
    m, k_total = adj.shape
    f = rhs.shape[1]
    if bm is None:
        bm = _pick_bm(m)
    bk = min(bk, ((k_total + 127) // 128) * 128)
    nm = m // bm
    nk = (k_total + bk - 1) // bk
    grid = (nm, nk)
    a_spec = pl.BlockSpec((bm, bk), lambda i, j: (i, j))
    b_spec = pl.BlockSpec((bk, f), lambda i, j: (j, 0))
    out_spec = pl.BlockSpec((bm, f), lambda i, j: (i, 0))
    scratch = [pltpu.VMEM((bm, f), jnp.float32)]
    params = pltpu.CompilerParams(
        dimension_semantics=("parallel", "arbitrary"),
    )
    if w is None:
        body = functools.partial(_matmul_kernel, k_total=k_total, bk=bk, nk=nk)
        return pl.pallas_call(
            body,
            grid=grid,
            in_specs=[a_spec, b_spec],
            out_specs=out_spec,
            out_shape=jax.ShapeDtypeStruct((m, f), jnp.float32),
            scratch_shapes=scratch,
            compiler_params=params,
        )(adj, rhs)
    nh = w.shape[0]
    w_spec = pl.BlockSpec((nh, f), lambda i, j: (0, 0))
    bias_spec = pl.BlockSpec((1, nh), lambda i, j: (0, 0))
    out_spec = pl.BlockSpec((bm, nh), lambda i, j: (i, 0))
    scratch = [pltpu.VMEM((bm, f), jnp.float32)]
    body = functools.partial(_matmul_linear_kernel, k_total=k_total, bk=bk,
                             nk=nk)
    return pl.pallas_call(
        body,
        grid=grid,
        in_specs=[a_spec, b_spec, w_spec, bias_spec],
        out_specs=out_spec,
        out_shape=jax.ShapeDtypeStruct((m, nh), jnp.float32),
        scratch_shapes=scratch,
        compiler_params=params,
    )(adj, rhs, w, bias.reshape(1, nh))


def kernel(x, adj, W, b):
    h = _propagate(adj, x)
    return _propagate(adj, h, w=W, bias=b)


# speedup vs baseline: 1.0066x; 1.0066x over previous
import functools

import jax
import jax.numpy as jnp
from jax.experimental import pallas as pl
from jax.experimental.pallas import tpu as pltpu


def _dot(a, b):
    return jnp.dot(a.astype(jnp.bfloat16), b.astype(jnp.bfloat16),
                   preferred_element_type=jnp.float32)


def _masked(a, b, rem, bk):
    col_ids = jax.lax.broadcasted_iota(jnp.int32, (1, bk), 1)
    row_ids = jax.lax.broadcasted_iota(jnp.int32, (bk, 1), 0)
    return jnp.where(col_ids < rem, a, 0.0), jnp.where(row_ids < rem, b, 0.0)


def _matmul_kernel(a_ref, b_ref, o_ref, *, k_total, bk, nk):
    k = pl.program_id(1)

    @pl.when(k == 0)
    def _init():
        o_ref[...] = jnp.zeros_like(o_ref)

    @pl.when(k < nk - 1)
    def _body():
        o_ref[...] += _dot(a_ref[...], b_ref[...])

    @pl.when(k == nk - 1)
    def _last():
        a, b = _masked(a_ref[...], b_ref[...], k_total - k * bk, bk)
        o_ref[...] += _dot(a, b)


def _matmul_linear_kernel(a_ref, b_ref, w_ref, bias_ref, o_ref, acc_ref,
                          *, k_total, bk, nk):
    k = pl.program_id(1)

    @pl.when(k == 0)
    def _init():
        acc_ref[...] = jnp.zeros_like(acc_ref)

    @pl.when(k < nk - 1)
    def _body():
        acc_ref[...] += _dot(a_ref[...], b_ref[...])

    @pl.when(k == nk - 1)
    def _last():
        a, b = _masked(a_ref[...], b_ref[...], k_total - k * bk, bk)
        acc = acc_ref[...] + _dot(a, b)
        out = jax.lax.dot_general(
            acc.astype(jnp.bfloat16), w_ref[...].astype(jnp.bfloat16),
            (((1,), (1,)), ((), ())),
            preferred_element_type=jnp.float32,
        )
        o_ref[...] = out + bias_ref[...]


def _pick_bm(m):
    for cand in (2000, 1024, 512, 256, 128, 64, 32, 16, 8):
        if m % cand == 0:
            return cand
    return m


def _propagate(adj, rhs, w=None, bias=None, bm=None, bk=2048):
    m, k_total = adj.shape
    f = rhs.shape[1]
    if bm is None:
        bm = _pick_bm(m)
    bk = min(bk, ((k_total + 127) // 128) * 128)
    nm = m // bm
    nk = (k_total + bk - 1) // bk
    grid = (nm, nk)
    a_spec = pl.BlockSpec((bm, bk), lambda i, j: (i, j))
    b_spec = pl.BlockSpec((bk, f), lambda i, j: (j, 0))
    params = pltpu.CompilerParams(
        dimension_semantics=("parallel", "arbitrary"),
    )
    if w is None:
        out_spec = pl.BlockSpec((bm, f), lambda i, j: (i, 0))
        body = functools.partial(_matmul_kernel, k_total=k_total, bk=bk, nk=nk)
        return pl.pallas_call(
            body,
            grid=grid,
            in_specs=[a_spec, b_spec],
            out_specs=out_spec,
            out_shape=jax.ShapeDtypeStruct((m, f), jnp.float32),
            compiler_params=params,
        )(adj, rhs)
    nh = w.shape[0]
    w_spec = pl.BlockSpec((nh, f), lambda i, j: (0, 0))
    bias_spec = pl.BlockSpec((1, nh), lambda i, j: (0, 0))
    out_spec = pl.BlockSpec((bm, nh), lambda i, j: (i, 0))
    body = functools.partial(_matmul_linear_kernel, k_total=k_total, bk=bk,
                             nk=nk)
    return pl.pallas_call(
        body,
        grid=grid,
        in_specs=[a_spec, b_spec, w_spec, bias_spec],
        out_specs=out_spec,
        out_shape=jax.ShapeDtypeStruct((m, nh), jnp.float32),
        scratch_shapes=[pltpu.VMEM((bm, f), jnp.float32)],
        compiler_params=params,
    )(adj, rhs, w, bias.reshape(1, nh))


def kernel(x, adj, W, b):
    h = _propagate(adj, x)
    return _propagate(adj, h, w=W, bias=b)


# speedup vs baseline: 1.1064x; 1.0992x over previous
import functools

import jax
import jax.numpy as jnp
from jax.experimental import pallas as pl
from jax.experimental.pallas import tpu as pltpu


def _strip_kernel(a_ref, b_ref, o_ref):
    h = jnp.dot(a_ref[...].astype(jnp.bfloat16), b_ref[...],
                preferred_element_type=jnp.float32)
    o_ref[...] = h.astype(jnp.bfloat16)


def _strip_linear_kernel(a_ref, b_ref, w_ref, bias_ref, o_ref):
    h = jnp.dot(a_ref[...].astype(jnp.bfloat16), b_ref[...],
                preferred_element_type=jnp.float32)
    out = jax.lax.dot_general(
        h.astype(jnp.bfloat16), w_ref[...], (((1,), (1,)), ((), ())),
        preferred_element_type=jnp.float32,
    )
    o_ref[...] = out + bias_ref[...]


def _pick_bm(m):
    for cand in (400, 256, 128, 64, 32, 16, 8):
        if m % cand == 0:
            return cand
    return m


def _propagate(adj, rhs_bf16, w=None, bias=None, bm=None):
    m, k_total = adj.shape
    f = rhs_bf16.shape[1]
    if bm is None:
        bm = _pick_bm(m)
    grid = (m // bm,)
    a_spec = pl.BlockSpec((bm, k_total), lambda i: (i, 0))
    b_spec = pl.BlockSpec((k_total, f), lambda i: (0, 0))
    params = pltpu.CompilerParams(
        dimension_semantics=("arbitrary",),
    )
    if w is None:
        return pl.pallas_call(
            _strip_kernel,
            grid=grid,
            in_specs=[a_spec, b_spec],
            out_specs=pl.BlockSpec((bm, f), lambda i: (i, 0)),
            out_shape=jax.ShapeDtypeStruct((m, f), jnp.bfloat16),
            compiler_params=params,
        )(adj, rhs_bf16)
    nh = w.shape[0]
    return pl.pallas_call(
        _strip_linear_kernel,
        grid=grid,
        in_specs=[a_spec, b_spec,
                  pl.BlockSpec((nh, f), lambda i: (0, 0)),
                  pl.BlockSpec((1, nh), lambda i: (0, 0))],
        out_specs=pl.BlockSpec((bm, nh), lambda i: (i, 0)),
        out_shape=jax.ShapeDtypeStruct((m, nh), jnp.float32),
        compiler_params=params,
    )(adj, rhs_bf16, w, bias.reshape(1, nh))


def kernel(x, adj, W, b):
    h = _propagate(adj, x.astype(jnp.bfloat16))
    return _propagate(adj, h, w=W.astype(jnp.bfloat16), bias=b)
